# baseline (speedup 1.0000x reference)
import numpy as np

import jax
import jax.numpy as jnp
from jax.experimental import pallas as pl
from jax.experimental.pallas import tpu as pltpu


def _band_np(W):
    D = np.zeros((3, W, W), np.float32)
    for kx in range(3):
        for wo in range(W):
            wi = wo + kx - 1
            if 0 <= wi < W:
                D[kx, wi, wo] = 1.0
    return D


def _pool_np(W, C):
    Dp = np.zeros((W, W // 2), np.float32)
    for wi in range(W):
        Dp[wi, wi // 2] = 0.25
    return np.kron(np.eye(C, dtype=np.float32), Dp)


def _conv_wmats(w, W):
    Ci, Co = w.shape[2], w.shape[3]
    D = jnp.asarray(_band_np(W))
    m = jnp.einsum("xiw,kxab->kaibw", D, w)
    return m.reshape(3, Ci * W, Co * W).astype(jnp.bfloat16)


def _bias_row(b, W):
    return jnp.repeat(b, W).reshape(1, -1).astype(jnp.float32)


def _conv3(a_bf16_3d, wstack, BB, H, K):
    acc = None
    for ky in range(3):
        opnd = a_bf16_3d[ky:ky + H].reshape(H * BB, K)
        d = jnp.dot(opnd, wstack[ky], preferred_element_type=jnp.float32)
        acc = d if acc is None else acc + d
    return acc


def _pad_rows(a_f32_2d, BB, H, K):
    a3 = a_f32_2d.astype(jnp.bfloat16).reshape(H, BB, K)
    z = jnp.zeros((1, BB, K), jnp.bfloat16)
    return jnp.concatenate([z, a3, z], axis=0)


def _pool(a_f32_2d, p_ref, BB, H, K):
    a4 = a_f32_2d.reshape(H // 2, 2, BB, K)
    s = (a4[:, 0] + a4[:, 1]).astype(jnp.bfloat16).reshape((H // 2) * BB, K)
    return jnp.dot(s, p_ref[...], preferred_element_type=jnp.float32)


def _resblock(x_f32_2d, wa, ba, wb, bb, BB, H, K):
    hp = _pad_rows(x_f32_2d, BB, H, K)
    h = jnp.maximum(_conv3(hp, wa, BB, H, K) + ba[...], 0.0)
    hp2 = _pad_rows(h, BB, H, K)
    o = _conv3(hp2, wb, BB, H, K) + bb[...] + x_f32_2d
    return jnp.maximum(o, 0.0)


def _forward(x, refs, G):
    (w1, b1, p1, r1aw, r1ab, r1bw, r1bb, w2, b2, p2,
     r2aw, r2ab, r2bw, r2bb, w3, b3, p3, r3aw, r3ab, r3bw, r3bb,
     w4, b4, p4, fc1w, fc1b, fc2w, fc2b) = refs

    z = jnp.zeros((1, G, 96), jnp.bfloat16)
    xp = jnp.concatenate([z, x, z], axis=0)
    a = jnp.maximum(_conv3(xp, w1, G, 32, 96) + b1[...], 0.0)
    a = _pool(a, p1, G, 32, 512)

    a = _resblock(a, r1aw, r1ab, r1bw, r1bb, G, 16, 256)

    ap = _pad_rows(a, G, 16, 256)
    a = jnp.maximum(_conv3(ap, w2, G, 16, 256) + b2[...], 0.0)
    a = _pool(a, p2, G, 16, 512)

    a = _resblock(a, r2aw, r2ab, r2bw, r2bb, G, 8, 256)

    ap = _pad_rows(a, G, 8, 256)
    a = jnp.maximum(_conv3(ap, w3, G, 8, 256) + b3[...], 0.0)
    a = _pool(a, p3, G, 8, 512)

    a = _resblock(a, r3aw, r3ab, r3bw, r3bb, G, 4, 256)

    ap = _pad_rows(a, G, 4, 256)
    a = jnp.maximum(_conv3(ap, w4, G, 4, 256) + b4[...], 0.0)
    a = _pool(a, p4, G, 4, 512)

    f = a.astype(jnp.bfloat16).reshape(2, G, 256)
    h = (jnp.dot(f[0], fc1w[0], preferred_element_type=jnp.float32)
         + jnp.dot(f[1], fc1w[1], preferred_element_type=jnp.float32)
         + fc1b[...])
    h = jnp.maximum(h, 0.0).astype(jnp.bfloat16)

    return jnp.dot(h, fc2w[...], preferred_element_type=jnp.float32) + fc2b[...]


def _make_net_kernel(BB, NSUB):
    G = BB // NSUB

    def _net_kernel(x_ref, *refs_and_out):
        refs, o_ref = refs_and_out[:-1], refs_and_out[-1]
        x = x_ref[...]
        for s in range(NSUB):
            o = _forward(x[:, s * G:(s + 1) * G, :], refs, G)
            o_ref[s * G:(s + 1) * G, :] = o.astype(o_ref.dtype)

    return _net_kernel


def kernel(x, conv1_w, conv1_b, res1_w1, res1_b1, res1_w2, res1_b2,
           conv2_w, conv2_b, res2_w1, res2_b1, res2_w2, res2_b2,
           conv3_w, conv3_b, res3_w1, res3_b1, res3_w2, res3_b2,
           conv4_w, conv4_b, fc1_w, fc1_b, fc2_w, fc2_b):
    B = x.shape[0]
    BB = next(b for b in (128, 64, 32, 16, 8) if B % b == 0)
    NSUB = 1

    xr = (jnp.transpose(x.astype(jnp.bfloat16), (2, 0, 1, 3))
          .reshape(32, B, 96))

    inputs = [
        xr,
        _conv_wmats(conv1_w, 32), _bias_row(conv1_b, 32),
        jnp.asarray(_pool_np(32, 16), jnp.bfloat16),
        _conv_wmats(res1_w1, 16), _bias_row(res1_b1, 16),
        _conv_wmats(res1_w2, 16), _bias_row(res1_b2, 16),
        _conv_wmats(conv2_w, 16), _bias_row(conv2_b, 16),
        jnp.asarray(_pool_np(16, 32), jnp.bfloat16),
        _conv_wmats(res2_w1, 8), _bias_row(res2_b1, 8),
        _conv_wmats(res2_w2, 8), _bias_row(res2_b2, 8),
        _conv_wmats(conv3_w, 8), _bias_row(conv3_b, 8),
        jnp.asarray(_pool_np(8, 64), jnp.bfloat16),
        _conv_wmats(res3_w1, 4), _bias_row(res3_b1, 4),
        _conv_wmats(res3_w2, 4), _bias_row(res3_b2, 4),
        _conv_wmats(conv4_w, 4), _bias_row(conv4_b, 4),
        jnp.asarray(_pool_np(4, 128), jnp.bfloat16),
        fc1_w.reshape(2, 2, 128, 256).transpose(0, 2, 1, 3)
             .reshape(2, 256, 256).astype(jnp.bfloat16),
        fc1_b.reshape(1, 256).astype(jnp.float32),
        fc2_w.astype(jnp.bfloat16),
        fc2_b.reshape(1, 10).astype(jnp.float32),
    ]

    def _full(a):
        nd = a.ndim
        return pl.BlockSpec(a.shape, lambda i, _n=nd: (0,) * _n)

    in_specs = [pl.BlockSpec((32, BB, 96), lambda i: (0, i, 0))]
    in_specs += [_full(a) for a in inputs[1:]]

    out = pl.pallas_call(
        _make_net_kernel(BB, NSUB),
        out_shape=jax.ShapeDtypeStruct((B, 10), x.dtype),
        grid_spec=pl.GridSpec(
            grid=(B // BB,),
            in_specs=in_specs,
            out_specs=pl.BlockSpec((BB, 10), lambda i: (i, 0)),
        ),
        compiler_params=pltpu.CompilerParams(
            dimension_semantics=("parallel",),
            vmem_limit_bytes=100 * 1024 * 1024,
            allow_input_fusion=[True] + [False] * (len(inputs) - 1),
        ),
    )(*inputs)
    return out


# speedup vs baseline: 1.5426x; 1.5426x over previous
import numpy as np

import jax
import jax.numpy as jnp
from jax.experimental import pallas as pl
from jax.experimental.pallas import tpu as pltpu


def _band_np(W):
    D = np.zeros((3, W, W), np.float32)
    for kx in range(3):
        for wo in range(W):
            wi = wo + kx - 1
            if 0 <= wi < W:
                D[kx, wi, wo] = 1.0
    return D


def _pool_np(W, C):
    Dp = np.zeros((W, W // 2), np.float32)
    for wi in range(W):
        Dp[wi, wi // 2] = 0.25
    return np.kron(Dp, np.eye(C, dtype=np.float32))


def _conv_wmats(w, W):
    Ci, Co = w.shape[2], w.shape[3]
    D = jnp.asarray(_band_np(W))
    m = jnp.einsum("xiw,kxab->kiawb", D, w)
    return m.reshape(3, W * Ci, W * Co).astype(jnp.bfloat16)


def _bias_row(b, W):
    return jnp.tile(b, W).reshape(1, -1).astype(jnp.float32)


def _conv3(a_bf16_3d, wstack, BB, H, K):
    acc = None
    for ky in range(3):
        opnd = a_bf16_3d[ky:ky + H].reshape(H * BB, K)
        d = jnp.dot(opnd, wstack[ky], preferred_element_type=jnp.float32)
        acc = d if acc is None else acc + d
    return acc


def _pad_rows(a_f32_2d, BB, H, K):
    a3 = a_f32_2d.astype(jnp.bfloat16).reshape(H, BB, K)
    z = jnp.zeros((1, BB, K), jnp.bfloat16)
    return jnp.concatenate([z, a3, z], axis=0)


def _pool(a_f32_2d, p_ref, BB, H, K):
    a4 = a_f32_2d.reshape(H // 2, 2, BB, K)
    s = (a4[:, 0] + a4[:, 1]).astype(jnp.bfloat16).reshape((H // 2) * BB, K)
    return jnp.dot(s, p_ref[...], preferred_element_type=jnp.float32)


def _resblock(x_f32_2d, wa, ba, wb, bb, BB, H, K):
    hp = _pad_rows(x_f32_2d, BB, H, K)
    h = jnp.maximum(_conv3(hp, wa, BB, H, K) + ba[...], 0.0)
    hp2 = _pad_rows(h, BB, H, K)
    o = _conv3(hp2, wb, BB, H, K) + bb[...] + x_f32_2d
    return jnp.maximum(o, 0.0)


def _forward(x, refs, G):
    (w1, b1, p1, r1aw, r1ab, r1bw, r1bb, w2, b2, p2,
     r2aw, r2ab, r2bw, r2bb, w3, b3, p3, r3aw, r3ab, r3bw, r3bb,
     w4, b4, p4, fc1w, fc1b, fc2w, fc2b) = refs

    z = jnp.zeros((1, G, 96), jnp.bfloat16)
    xp = jnp.concatenate([z, x, z], axis=0)
    a = jnp.maximum(_conv3(xp, w1, G, 32, 96) + b1[...], 0.0)
    a = _pool(a, p1, G, 32, 512)

    a = _resblock(a, r1aw, r1ab, r1bw, r1bb, G, 16, 256)

    ap = _pad_rows(a, G, 16, 256)
    a = jnp.maximum(_conv3(ap, w2, G, 16, 256) + b2[...], 0.0)
    a = _pool(a, p2, G, 16, 512)

    a = _resblock(a, r2aw, r2ab, r2bw, r2bb, G, 8, 256)

    ap = _pad_rows(a, G, 8, 256)
    a = jnp.maximum(_conv3(ap, w3, G, 8, 256) + b3[...], 0.0)
    a = _pool(a, p3, G, 8, 512)

    a = _resblock(a, r3aw, r3ab, r3bw, r3bb, G, 4, 256)

    ap = _pad_rows(a, G, 4, 256)
    a = jnp.maximum(_conv3(ap, w4, G, 4, 256) + b4[...], 0.0)
    a = _pool(a, p4, G, 4, 512)

    f = a.astype(jnp.bfloat16).reshape(2, G, 256)
    h = (jnp.dot(f[0], fc1w[0], preferred_element_type=jnp.float32)
         + jnp.dot(f[1], fc1w[1], preferred_element_type=jnp.float32)
         + fc1b[...])
    h = jnp.maximum(h, 0.0).astype(jnp.bfloat16)

    return jnp.dot(h, fc2w[...], preferred_element_type=jnp.float32) + fc2b[...]


def _make_net_kernel(BB, NSUB):
    G = BB // NSUB

    def _net_kernel(x_ref, *refs_and_out):
        refs, o_ref = refs_and_out[:-1], refs_and_out[-1]
        x = x_ref[...]
        for s in range(NSUB):
            o = _forward(x[:, s * G:(s + 1) * G, :], refs, G)
            o_ref[s * G:(s + 1) * G, :] = o.astype(o_ref.dtype)

    return _net_kernel


def kernel(x, conv1_w, conv1_b, res1_w1, res1_b1, res1_w2, res1_b2,
           conv2_w, conv2_b, res2_w1, res2_b1, res2_w2, res2_b2,
           conv3_w, conv3_b, res3_w1, res3_b1, res3_w2, res3_b2,
           conv4_w, conv4_b, fc1_w, fc1_b, fc2_w, fc2_b):
    B = x.shape[0]
    BB = next(b for b in (256, 128, 64, 32, 16, 8) if B % b == 0)
    NSUB = 1

    xr = (jnp.transpose(x.astype(jnp.bfloat16), (2, 0, 3, 1))
          .reshape(32, B, 96))

    inputs = [
        xr,
        _conv_wmats(conv1_w, 32), _bias_row(conv1_b, 32),
        jnp.asarray(_pool_np(32, 16), jnp.bfloat16),
        _conv_wmats(res1_w1, 16), _bias_row(res1_b1, 16),
        _conv_wmats(res1_w2, 16), _bias_row(res1_b2, 16),
        _conv_wmats(conv2_w, 16), _bias_row(conv2_b, 16),
        jnp.asarray(_pool_np(16, 32), jnp.bfloat16),
        _conv_wmats(res2_w1, 8), _bias_row(res2_b1, 8),
        _conv_wmats(res2_w2, 8), _bias_row(res2_b2, 8),
        _conv_wmats(conv3_w, 8), _bias_row(conv3_b, 8),
        jnp.asarray(_pool_np(8, 64), jnp.bfloat16),
        _conv_wmats(res3_w1, 4), _bias_row(res3_b1, 4),
        _conv_wmats(res3_w2, 4), _bias_row(res3_b2, 4),
        _conv_wmats(conv4_w, 4), _bias_row(conv4_b, 4),
        jnp.asarray(_pool_np(4, 128), jnp.bfloat16),
        fc1_w.reshape(2, 256, 256).astype(jnp.bfloat16),
        fc1_b.reshape(1, 256).astype(jnp.float32),
        fc2_w.astype(jnp.bfloat16),
        fc2_b.reshape(1, 10).astype(jnp.float32),
    ]

    def _full(a):
        nd = a.ndim
        return pl.BlockSpec(a.shape, lambda i, _n=nd: (0,) * _n)

    in_specs = [pl.BlockSpec((32, BB, 96), lambda i: (0, i, 0))]
    in_specs += [_full(a) for a in inputs[1:]]

    out = pl.pallas_call(
        _make_net_kernel(BB, NSUB),
        out_shape=jax.ShapeDtypeStruct((B, 10), x.dtype),
        grid_spec=pl.GridSpec(
            grid=(B // BB,),
            in_specs=in_specs,
            out_specs=pl.BlockSpec((BB, 10), lambda i: (i, 0)),
        ),
        compiler_params=pltpu.CompilerParams(
            dimension_semantics=("parallel",),
            vmem_limit_bytes=100 * 1024 * 1024,
            allow_input_fusion=[True] + [False] * (len(inputs) - 1),
        ),
    )(*inputs)
    return out


# speedup vs baseline: 1.5686x; 1.0169x over previous
import numpy as np

import jax
import jax.numpy as jnp
from jax.experimental import pallas as pl
from jax.experimental.pallas import tpu as pltpu


def _band_np(W):
    D = np.zeros((3, W, W), np.float32)
    for kx in range(3):
        for wo in range(W):
            wi = wo + kx - 1
            if 0 <= wi < W:
                D[kx, wi, wo] = 1.0
    return D


def _pool_np(W, C):
    Dp = np.zeros((W, W // 2), np.float32)
    for wi in range(W):
        Dp[wi, wi // 2] = 0.25
    return np.kron(Dp, np.eye(C, dtype=np.float32))


def _conv_wmats(w, W):
    Ci, Co = w.shape[2], w.shape[3]
    D = jnp.asarray(_band_np(W))
    m = jnp.einsum("xiw,kxab->kiawb", D, w)
    return m.reshape(3, W * Ci, W * Co).astype(jnp.bfloat16)


def _bias_row(b, W):
    return jnp.tile(b, W).reshape(1, -1).astype(jnp.float32)


def _conv3(a_bf16_3d, wstack, BB, H, K):
    acc = None
    for ky in range(3):
        opnd = a_bf16_3d[ky:ky + H].reshape(H * BB, K)
        d = jnp.dot(opnd, wstack[ky], preferred_element_type=jnp.float32)
        acc = d if acc is None else acc + d
    return acc


def _pad_rows(a_f32_2d, BB, H, K):
    a3 = a_f32_2d.astype(jnp.bfloat16).reshape(H, BB, K)
    z = jnp.zeros((1, BB, K), jnp.bfloat16)
    return jnp.concatenate([z, a3, z], axis=0)


def _pool(a_f32_2d, p_ref, BB, H, K):
    a4 = a_f32_2d.reshape(H // 2, 2, BB, K)
    s = (a4[:, 0] + a4[:, 1]).astype(jnp.bfloat16).reshape((H // 2) * BB, K)
    return jnp.dot(s, p_ref[...], preferred_element_type=jnp.float32)


def _resblock(x_f32_2d, wa, ba, wb, bb, BB, H, K):
    hp = _pad_rows(x_f32_2d, BB, H, K)
    h = jnp.maximum(_conv3(hp, wa, BB, H, K) + ba[...], 0.0)
    hp2 = _pad_rows(h, BB, H, K)
    o = _conv3(hp2, wb, BB, H, K) + bb[...] + x_f32_2d
    return jnp.maximum(o, 0.0)


def _forward(x, refs, G):
    (w1, b1, p1, r1aw, r1ab, r1bw, r1bb, w2, b2, p2,
     r2aw, r2ab, r2bw, r2bb, w3, b3, p3, r3aw, r3ab, r3bw, r3bb,
     w4, b4, p4, fc1w, fc1b, fc2w, fc2b) = refs

    z = jnp.zeros((1, G, 96), jnp.bfloat16)
    xp = jnp.concatenate([z, x, z], axis=0)
    a = jnp.maximum(_conv3(xp, w1, G, 32, 96) + b1[...], 0.0)
    a = _pool(a, p1, G, 32, 512)

    a = _resblock(a, r1aw, r1ab, r1bw, r1bb, G, 16, 256)

    ap = _pad_rows(a, G, 16, 256)
    a = jnp.maximum(_conv3(ap, w2, G, 16, 256) + b2[...], 0.0)
    a = _pool(a, p2, G, 16, 512)

    a = _resblock(a, r2aw, r2ab, r2bw, r2bb, G, 8, 256)

    ap = _pad_rows(a, G, 8, 256)
    a = jnp.maximum(_conv3(ap, w3, G, 8, 256) + b3[...], 0.0)
    a = _pool(a, p3, G, 8, 512)

    a = _resblock(a, r3aw, r3ab, r3bw, r3bb, G, 4, 256)

    ap = _pad_rows(a, G, 4, 256)
    a = jnp.maximum(_conv3(ap, w4, G, 4, 256) + b4[...], 0.0)
    a = _pool(a, p4, G, 4, 512)

    f = a.astype(jnp.bfloat16).reshape(2, G, 256)
    h = (jnp.dot(f[0], fc1w[0], preferred_element_type=jnp.float32)
         + jnp.dot(f[1], fc1w[1], preferred_element_type=jnp.float32)
         + fc1b[...])
    h = jnp.maximum(h, 0.0).astype(jnp.bfloat16)

    return jnp.dot(h, fc2w[...], preferred_element_type=jnp.float32) + fc2b[...]


def _make_net_kernel(BB, NSUB):
    G = BB // NSUB

    def _net_kernel(x_ref, *refs_and_out):
        refs, o_ref = refs_and_out[:-1], refs_and_out[-1]
        x = x_ref[...]
        for s in range(NSUB):
            o = _forward(x[:, s * G:(s + 1) * G, :], refs, G)
            o_ref[s * G:(s + 1) * G, :] = o.astype(o_ref.dtype)

    return _net_kernel


def kernel(x, conv1_w, conv1_b, res1_w1, res1_b1, res1_w2, res1_b2,
           conv2_w, conv2_b, res2_w1, res2_b1, res2_w2, res2_b2,
           conv3_w, conv3_b, res3_w1, res3_b1, res3_w2, res3_b2,
           conv4_w, conv4_b, fc1_w, fc1_b, fc2_w, fc2_b):
    B = x.shape[0]
    BB = next(b for b in (128, 64, 32, 16, 8) if B % b == 0)
    NSUB = 1

    xr = (jnp.transpose(x.astype(jnp.bfloat16), (2, 0, 3, 1))
          .reshape(32, B, 96))

    inputs = [
        xr,
        _conv_wmats(conv1_w, 32), _bias_row(conv1_b, 32),
        jnp.asarray(_pool_np(32, 16), jnp.bfloat16),
        _conv_wmats(res1_w1, 16), _bias_row(res1_b1, 16),
        _conv_wmats(res1_w2, 16), _bias_row(res1_b2, 16),
        _conv_wmats(conv2_w, 16), _bias_row(conv2_b, 16),
        jnp.asarray(_pool_np(16, 32), jnp.bfloat16),
        _conv_wmats(res2_w1, 8), _bias_row(res2_b1, 8),
        _conv_wmats(res2_w2, 8), _bias_row(res2_b2, 8),
        _conv_wmats(conv3_w, 8), _bias_row(conv3_b, 8),
        jnp.asarray(_pool_np(8, 64), jnp.bfloat16),
        _conv_wmats(res3_w1, 4), _bias_row(res3_b1, 4),
        _conv_wmats(res3_w2, 4), _bias_row(res3_b2, 4),
        _conv_wmats(conv4_w, 4), _bias_row(conv4_b, 4),
        jnp.asarray(_pool_np(4, 128), jnp.bfloat16),
        fc1_w.reshape(2, 256, 256).astype(jnp.bfloat16),
        fc1_b.reshape(1, 256).astype(jnp.float32),
        fc2_w.astype(jnp.bfloat16),
        fc2_b.reshape(1, 10).astype(jnp.float32),
    ]

    def _full(a):
        nd = a.ndim
        return pl.BlockSpec(a.shape, lambda i, _n=nd: (0,) * _n)

    in_specs = [pl.BlockSpec((32, BB, 96), lambda i: (0, i, 0))]
    in_specs += [_full(a) for a in inputs[1:]]

    out = pl.pallas_call(
        _make_net_kernel(BB, NSUB),
        out_shape=jax.ShapeDtypeStruct((B, 10), x.dtype),
        grid_spec=pl.GridSpec(
            grid=(B // BB,),
            in_specs=in_specs,
            out_specs=pl.BlockSpec((BB, 10), lambda i: (i, 0)),
        ),
        compiler_params=pltpu.CompilerParams(
            dimension_semantics=("parallel",),
            vmem_limit_bytes=100 * 1024 * 1024,
            allow_input_fusion=[True] + [False] * (len(inputs) - 1),
        ),
    )(*inputs)
    return out
